# baseline (speedup 1.0000x reference)
import functools

import jax
import jax.numpy as jnp
from jax import lax
from jax.experimental import pallas as pl
from jax.experimental.pallas import tpu as pltpu
from jax.experimental.pallas import tpu_sc as plsc

N = 10000
C = 128
NNZ = 320000
EPS = 1e-05

NC = 2
NS = 16
NW = NC * NS
E_W = NNZ // NW
B = 80
NB = E_W // B
ROW_T = 624


def _prologue_body(x_ref, wd_ref, wu_ref, ad_ref, au_ref,
                   xmd_ref, xmu_ref, sd_ref, td_ref, su_ref, tu_ref):
    xb = x_ref[...]
    xmd = jnp.dot(xb, wd_ref[...], preferred_element_type=jnp.float32)
    xmu = jnp.dot(xb, wu_ref[...], preferred_element_type=jnp.float32)
    xmd_ref[...] = xmd
    xmu_ref[...] = xmu
    ad = ad_ref[...]
    au = au_ref[...]
    sd_ref[...] = jnp.dot(xmd, ad[0:C, :], preferred_element_type=jnp.float32)
    td_ref[...] = jnp.dot(xmd, ad[C:2 * C, :], preferred_element_type=jnp.float32)
    su_ref[...] = jnp.dot(xmu, au[0:C, :], preferred_element_type=jnp.float32)
    tu_ref[...] = jnp.dot(xmu, au[C:2 * C, :], preferred_element_type=jnp.float32)


def _make_prologue():
    blk = 1000
    grid = (N // blk,)
    full = lambda shape: pl.BlockSpec(shape, lambda i: (0,) * len(shape))
    rowb = pl.BlockSpec((blk, C), lambda i: (i, 0))
    colb = pl.BlockSpec((blk, 1), lambda i: (i, 0))
    return pl.pallas_call(
        _prologue_body,
        grid=grid,
        in_specs=[rowb, full((C, C)), full((C, C)),
                  full((2 * C, 1)), full((2 * C, 1))],
        out_specs=[rowb, rowb, colb, colb, colb, colb],
        out_shape=[
            jax.ShapeDtypeStruct((N, C), jnp.float32),
            jax.ShapeDtypeStruct((N, C), jnp.float32),
            jax.ShapeDtypeStruct((N, 1), jnp.float32),
            jax.ShapeDtypeStruct((N, 1), jnp.float32),
            jax.ShapeDtypeStruct((N, 1), jnp.float32),
            jax.ShapeDtypeStruct((N, 1), jnp.float32),
        ],
    )


NSETS = 4
SKEW = 2


def _sc_body(xmd_hbm, xmu_hbm, srcd_hbm, tgtd_hbm, valsd_hbm,
             srcu_hbm, tgtu_hbm, valsu_hbm, sd_hbm, td_hbm, su_hbm, tu_hbm,
             p_out, *scr):
    sets = []
    for i in range(NSETS):
        o = 11 * i
        sets.append(dict(iS=scr[o], iT=scr[o + 1], v=scr[o + 2], w=scr[o + 3],
                         iT2=scr[o + 4], sb=scr[o + 5], tb=scr[o + 6],
                         rows=scr[o + 7], semI=scr[o + 8], semG=scr[o + 9],
                         semS=scr[o + 10]))
    acc = scr[11 * NSETS]
    rows = sets[0]["rows"]
    c = lax.axis_index("c")
    s = lax.axis_index("s")
    wid = c * NS + s
    base_e = wid * E_W

    zero16 = jnp.zeros((16,), jnp.float32)

    def _zrow(e, _):
        for j in range(C // 16):
            rows[e, pl.ds(j * 16, 16)] = zero16
        return 0
    lax.fori_loop(0, B, _zrow, 0)

    zbase = s * ROW_T
    for k in range(ROW_T // B):
        pltpu.sync_copy(rows, acc.at[pl.ds(zbase + k * B, B)])
    rem = ROW_T - (ROW_T // B) * B
    pltpu.sync_copy(rows.at[pl.ds(0, rem)],
                    acc.at[pl.ds(zbase + (ROW_T // B) * B, rem)])

    @pl.when(s == NS - 1)
    def _zero_tail():
        pltpu.sync_copy(rows.at[pl.ds(0, N - NS * ROW_T)],
                        acc.at[pl.ds(NS * ROW_T, N - NS * ROW_T)])

    plsc.subcore_barrier()

    NQ = NB // NSETS

    for (xm_hbm, src_hbm, tgt_hbm, vals_hbm, s_hbm, t_hbm) in (
            (xmd_hbm, srcd_hbm, tgtd_hbm, valsd_hbm, sd_hbm, td_hbm),
            (xmu_hbm, srcu_hbm, tgtu_hbm, valsu_hbm, su_hbm, tu_hbm)):

        def _prefetch(off, S):
            pltpu.async_copy(src_hbm.at[pl.ds(off, B)], S["iS"], S["semI"])
            pltpu.async_copy(tgt_hbm.at[pl.ds(off, B)], S["iT"], S["semI"])
            pltpu.async_copy(vals_hbm.at[pl.ds(off, B)], S["v"], S["semI"])

        def _G(off, S, wait_scatter):
            pltpu.make_async_copy(src_hbm.at[pl.ds(off, B)], S["iS"],
                                  S["semI"]).wait()
            pltpu.make_async_copy(tgt_hbm.at[pl.ds(off, B)], S["iT"],
                                  S["semI"]).wait()
            pltpu.make_async_copy(vals_hbm.at[pl.ds(off, B)], S["v"],
                                  S["semI"]).wait()
            if wait_scatter:
                pltpu.make_async_copy(S["rows"], acc.at[S["iT2"]],
                                      S["semS"]).wait()
            pltpu.async_copy(xm_hbm.at[S["iS"]], S["rows"], S["semG"])
            pltpu.async_copy(s_hbm.at[S["iS"]], S["sb"], S["semG"])
            pltpu.async_copy(t_hbm.at[S["iT"]], S["tb"], S["semG"])

        def _P(S, prefetch_off=None, guard=None):
            pltpu.make_async_copy(xm_hbm.at[S["iS"]], S["rows"],
                                  S["semG"]).wait()
            pltpu.make_async_copy(s_hbm.at[S["iS"]], S["sb"], S["semG"]).wait()
            pltpu.make_async_copy(t_hbm.at[S["iT"]], S["tb"], S["semG"]).wait()
            for j in range(B // 16):
                sl = pl.ds(j * 16, 16)
                a = S["sb"][sl] + S["tb"][sl]
                e = jnp.where(a > 0.0, a, jnp.exp(a) - 1.0)
                S["w"][sl] = S["v"][sl] * e
                S["iT2"][sl] = S["iT"][sl]
            if prefetch_off is not None:
                if guard is None:
                    _prefetch(prefetch_off, S)
                else:
                    @pl.when(guard)
                    def _():
                        _prefetch(prefetch_off, S)

            @plsc.parallel_loop(0, B, 1, unroll=8)
            def _scale(e2):
                wv = plsc.load_gather(
                    S["w"], [jnp.full((16,), e2, dtype=jnp.int32)])
                for j in range(C // 16):
                    S["rows"][e2, pl.ds(j * 16, 16)] = (
                        S["rows"][e2, pl.ds(j * 16, 16)] * wv)

            pltpu.async_copy(S["rows"], acc.at[S["iT2"]], S["semS"], add=True)

        for i in range(NSETS):
            _prefetch(base_e + i * B, sets[i])
        _G(base_e + 0 * B, sets[0], False)
        _G(base_e + 1 * B, sets[1], False)
        _G(base_e + 2 * B, sets[2], False)
        _P(sets[0], base_e + 4 * B)
        _G(base_e + 3 * B, sets[3], False)
        _P(sets[1], base_e + 5 * B)

        def _quad(q, _):
            for i in range(NSETS):
                b = 4 * q + i
                _G(base_e + b * B, sets[i], True)
                _P(sets[(i + SKEW) % NSETS], base_e + (b + SKEW) * B,
                   guard=b + SKEW <= NB - 1)
            return 0

        lax.fori_loop(1, NQ, _quad, 0)
        _G(base_e + (NB - 1) * B, sets[0], True)
        _P(sets[2])
        _P(sets[3])
        _P(sets[0])
        for i in range(NSETS):
            S = sets[i]
            pltpu.make_async_copy(S["rows"], acc.at[S["iT2"]],
                                  S["semS"]).wait()

    plsc.subcore_barrier()

    pltpu.sync_copy(acc.at[pl.ds(s * ROW_T, ROW_T)],
                    p_out.at[c, pl.ds(s * ROW_T, ROW_T)])

    @pl.when(s == NS - 1)
    def _write_tail():
        pltpu.sync_copy(acc.at[pl.ds(NS * ROW_T, N - NS * ROW_T)],
                        p_out.at[c, pl.ds(NS * ROW_T, N - NS * ROW_T)])


def _make_sc_kernel():
    mesh = plsc.VectorSubcoreMesh(core_axis_name="c", subcore_axis_name="s",
                                  num_cores=NC, num_subcores=NS)
    return pl.kernel(
        _sc_body,
        out_type=jax.ShapeDtypeStruct((NC, N, C), jnp.float32),
        mesh=mesh,
        compiler_params=pltpu.CompilerParams(needs_layout_passes=False),
        scratch_types=(
            [t for _ in range(NSETS) for t in (
                pltpu.VMEM((B,), jnp.int32),
                pltpu.VMEM((B,), jnp.int32),
                pltpu.VMEM((B,), jnp.float32),
                pltpu.VMEM((B,), jnp.float32),
                pltpu.VMEM((B,), jnp.int32),
                pltpu.VMEM((B,), jnp.float32),
                pltpu.VMEM((B,), jnp.float32),
                pltpu.VMEM((B, C), jnp.float32),
                pltpu.SemaphoreType.DMA,
                pltpu.SemaphoreType.DMA,
                pltpu.SemaphoreType.DMA,
            )]
            + [pltpu.VMEM_SHARED((N, C), jnp.float32)]
        ),
    )


def _epilogue_body(p_ref, x_ref, wi_ref, aw_ref, h_ref):
    xmi = jnp.dot(x_ref[...], wi_ref[...], preferred_element_type=jnp.float32)
    m = p_ref[0] + p_ref[1] + (1.0 + EPS) * xmi
    g = jnp.dot(m, aw_ref[...], preferred_element_type=jnp.float32)
    h_ref[...] = jax.nn.sigmoid(g) * jax.nn.sigmoid(m)


def _make_epilogue():
    blk = 1000
    grid = (N // blk,)
    return pl.pallas_call(
        _epilogue_body,
        grid=grid,
        in_specs=[pl.BlockSpec((NC, blk, C), lambda i: (0, i, 0)),
                  pl.BlockSpec((blk, C), lambda i: (i, 0)),
                  pl.BlockSpec((C, C), lambda i: (0, 0)),
                  pl.BlockSpec((C, 1), lambda i: (0, 0))],
        out_specs=pl.BlockSpec((blk, C), lambda i: (i, 0)),
        out_shape=jax.ShapeDtypeStruct((N, C), jnp.float32),
    )


@jax.jit
def kernel(x_1, down_laplacian_indices, down_laplacian_values,
           up_laplacian_indices, up_laplacian_values,
           W_down, att_down, W_up, att_up, W_id, att_weight):
    xmd, xmu, sd, td, su, tu = _make_prologue()(
        x_1, W_down, W_up, att_down, att_up)

    tgt_d = down_laplacian_indices[0]
    src_d = down_laplacian_indices[1]
    tgt_u = up_laplacian_indices[0]
    src_u = up_laplacian_indices[1]

    partials = _make_sc_kernel()(
        xmd, xmu,
        src_d, tgt_d, down_laplacian_values,
        src_u, tgt_u, up_laplacian_values,
        sd.reshape(N), td.reshape(N), su.reshape(N), tu.reshape(N))

    return _make_epilogue()(partials, x_1, W_id, att_weight)


# speedup vs baseline: 1.0100x; 1.0100x over previous
import jax
import jax.numpy as jnp
from jax import lax
from jax.experimental import pallas as pl
from jax.experimental.pallas import tpu as pltpu
from jax.experimental.pallas import tpu_sc as plsc

N = 10000
C = 128
NNZ = 320000
EPS = 1e-05

NC = 2
NS = 16
NW = NC * NS
E_W = NNZ // NW
B = 80
NB = E_W // B
ROW_T = 624


def _prologue_body(x_ref, wd_ref, wu_ref, ad_ref, au_ref,
                   xmd_ref, xmu_ref, sd_ref, td_ref, su_ref, tu_ref):
    xb = x_ref[...]
    xmd = jnp.dot(xb, wd_ref[...], preferred_element_type=jnp.float32)
    xmu = jnp.dot(xb, wu_ref[...], preferred_element_type=jnp.float32)
    xmd_ref[...] = xmd
    xmu_ref[...] = xmu
    ad = ad_ref[...]
    au = au_ref[...]
    sd_ref[...] = jnp.dot(xmd, ad[0:C, :], preferred_element_type=jnp.float32)
    td_ref[...] = jnp.dot(xmd, ad[C:2 * C, :], preferred_element_type=jnp.float32)
    su_ref[...] = jnp.dot(xmu, au[0:C, :], preferred_element_type=jnp.float32)
    tu_ref[...] = jnp.dot(xmu, au[C:2 * C, :], preferred_element_type=jnp.float32)


def _make_prologue():
    blk = 1000
    grid = (N // blk,)
    full = lambda shape: pl.BlockSpec(shape, lambda i: (0,) * len(shape))
    rowb = pl.BlockSpec((blk, C), lambda i: (i, 0))
    colb = pl.BlockSpec((blk, 1), lambda i: (i, 0))
    return pl.pallas_call(
        _prologue_body,
        grid=grid,
        in_specs=[rowb, full((C, C)), full((C, C)),
                  full((2 * C, 1)), full((2 * C, 1))],
        out_specs=[rowb, rowb, colb, colb, colb, colb],
        out_shape=[
            jax.ShapeDtypeStruct((N, C), jnp.float32),
            jax.ShapeDtypeStruct((N, C), jnp.float32),
            jax.ShapeDtypeStruct((N, 1), jnp.float32),
            jax.ShapeDtypeStruct((N, 1), jnp.float32),
            jax.ShapeDtypeStruct((N, 1), jnp.float32),
            jax.ShapeDtypeStruct((N, 1), jnp.float32),
        ],
    )


NSETS = 4
SKEW = 2


def _sc_body(xmd_hbm, xmu_hbm, srcd_hbm, tgtd_hbm, valsd_hbm,
             srcu_hbm, tgtu_hbm, valsu_hbm, sd_hbm, td_hbm, su_hbm, tu_hbm,
             p_out, *scr):
    sets = []
    for i in range(NSETS):
        o = 11 * i
        sets.append(dict(iS=scr[o], iT=scr[o + 1], v=scr[o + 2], w=scr[o + 3],
                         iT2=scr[o + 4], sb=scr[o + 5], tb=scr[o + 6],
                         rows=scr[o + 7], semI=scr[o + 8], semG=scr[o + 9],
                         semS=scr[o + 10]))
    acc = scr[11 * NSETS]
    rows = sets[0]["rows"]
    c = lax.axis_index("c")
    s = lax.axis_index("s")
    wid = c * NS + s
    base_e = wid * E_W

    zero16 = jnp.zeros((16,), jnp.float32)

    def _zrow(e, _):
        for j in range(C // 16):
            rows[e, pl.ds(j * 16, 16)] = zero16
        return 0
    lax.fori_loop(0, B, _zrow, 0)

    zbase = s * ROW_T
    for k in range(ROW_T // B):
        pltpu.sync_copy(rows, acc.at[pl.ds(zbase + k * B, B)])
    rem = ROW_T - (ROW_T // B) * B
    pltpu.sync_copy(rows.at[pl.ds(0, rem)],
                    acc.at[pl.ds(zbase + (ROW_T // B) * B, rem)])

    @pl.when(s == NS - 1)
    def _zero_tail():
        pltpu.sync_copy(rows.at[pl.ds(0, N - NS * ROW_T)],
                        acc.at[pl.ds(NS * ROW_T, N - NS * ROW_T)])

    plsc.subcore_barrier()

    NQ = NB // NSETS

    for (xm_hbm, src_hbm, tgt_hbm, vals_hbm, s_hbm, t_hbm) in (
            (xmd_hbm, srcd_hbm, tgtd_hbm, valsd_hbm, sd_hbm, td_hbm),
            (xmu_hbm, srcu_hbm, tgtu_hbm, valsu_hbm, su_hbm, tu_hbm)):

        def _prefetch(off, S):
            pltpu.async_copy(src_hbm.at[pl.ds(off, B)], S["iS"], S["semI"])
            pltpu.async_copy(tgt_hbm.at[pl.ds(off, B)], S["iT"], S["semI"])
            pltpu.async_copy(vals_hbm.at[pl.ds(off, B)], S["v"], S["semI"])

        def _G(off, S, wait_scatter):
            pltpu.make_async_copy(src_hbm.at[pl.ds(off, B)], S["iS"],
                                  S["semI"]).wait()
            pltpu.make_async_copy(tgt_hbm.at[pl.ds(off, B)], S["iT"],
                                  S["semI"]).wait()
            pltpu.make_async_copy(vals_hbm.at[pl.ds(off, B)], S["v"],
                                  S["semI"]).wait()
            if wait_scatter:
                pltpu.make_async_copy(S["rows"], acc.at[S["iT2"]],
                                      S["semS"]).wait()
            pltpu.async_copy(xm_hbm.at[S["iS"]], S["rows"], S["semG"])
            pltpu.async_copy(s_hbm.at[S["iS"]], S["sb"], S["semG"])
            pltpu.async_copy(t_hbm.at[S["iT"]], S["tb"], S["semG"])

        def _P(S, prefetch_off=None, guard=None):
            pltpu.make_async_copy(xm_hbm.at[S["iS"]], S["rows"],
                                  S["semG"]).wait()
            pltpu.make_async_copy(s_hbm.at[S["iS"]], S["sb"], S["semG"]).wait()
            pltpu.make_async_copy(t_hbm.at[S["iT"]], S["tb"], S["semG"]).wait()
            for j in range(B // 16):
                sl = pl.ds(j * 16, 16)
                a = S["sb"][sl] + S["tb"][sl]
                e = jnp.where(a > 0.0, a, jnp.exp(a) - 1.0)
                S["w"][sl] = S["v"][sl] * e
                S["iT2"][sl] = S["iT"][sl]
            if prefetch_off is not None:
                if guard is None:
                    _prefetch(prefetch_off, S)
                else:
                    @pl.when(guard)
                    def _():
                        _prefetch(prefetch_off, S)

            @plsc.parallel_loop(0, B, 1, unroll=4)
            def _scale(e2):
                wv = plsc.load_gather(
                    S["w"], [jnp.full((16,), e2, dtype=jnp.int32)])
                for j in range(C // 16):
                    S["rows"][e2, pl.ds(j * 16, 16)] = (
                        S["rows"][e2, pl.ds(j * 16, 16)] * wv)

            pltpu.async_copy(S["rows"], acc.at[S["iT2"]], S["semS"], add=True)

        for i in range(NSETS):
            _prefetch(base_e + i * B, sets[i])
        _G(base_e + 0 * B, sets[0], False)
        _G(base_e + 1 * B, sets[1], False)
        _G(base_e + 2 * B, sets[2], False)
        _P(sets[0], base_e + 4 * B)
        _G(base_e + 3 * B, sets[3], False)
        _P(sets[1], base_e + 5 * B)

        def _quad(q, _):
            for i in range(NSETS):
                b = 4 * q + i
                _G(base_e + b * B, sets[i], True)
                _P(sets[(i + SKEW) % NSETS], base_e + (b + SKEW) * B,
                   guard=b + SKEW <= NB - 1)
            return 0

        lax.fori_loop(1, NQ, _quad, 0)
        _G(base_e + (NB - 1) * B, sets[0], True)
        _P(sets[2])
        _P(sets[3])
        _P(sets[0])
        for i in range(NSETS):
            S = sets[i]
            pltpu.make_async_copy(S["rows"], acc.at[S["iT2"]],
                                  S["semS"]).wait()

    plsc.subcore_barrier()

    pltpu.sync_copy(acc.at[pl.ds(s * ROW_T, ROW_T)],
                    p_out.at[c, pl.ds(s * ROW_T, ROW_T)])

    @pl.when(s == NS - 1)
    def _write_tail():
        pltpu.sync_copy(acc.at[pl.ds(NS * ROW_T, N - NS * ROW_T)],
                        p_out.at[c, pl.ds(NS * ROW_T, N - NS * ROW_T)])


def _make_sc_kernel():
    mesh = plsc.VectorSubcoreMesh(core_axis_name="c", subcore_axis_name="s",
                                  num_cores=NC, num_subcores=NS)
    return pl.kernel(
        _sc_body,
        out_type=jax.ShapeDtypeStruct((NC, N, C), jnp.float32),
        mesh=mesh,
        compiler_params=pltpu.CompilerParams(needs_layout_passes=False),
        scratch_types=(
            [t for _ in range(NSETS) for t in (
                pltpu.VMEM((B,), jnp.int32),
                pltpu.VMEM((B,), jnp.int32),
                pltpu.VMEM((B,), jnp.float32),
                pltpu.VMEM((B,), jnp.float32),
                pltpu.VMEM((B,), jnp.int32),
                pltpu.VMEM((B,), jnp.float32),
                pltpu.VMEM((B,), jnp.float32),
                pltpu.VMEM((B, C), jnp.float32),
                pltpu.SemaphoreType.DMA,
                pltpu.SemaphoreType.DMA,
                pltpu.SemaphoreType.DMA,
            )]
            + [pltpu.VMEM_SHARED((N, C), jnp.float32)]
        ),
    )


def _epilogue_body(p_ref, x_ref, wi_ref, aw_ref, h_ref):
    xmi = jnp.dot(x_ref[...], wi_ref[...], preferred_element_type=jnp.float32)
    m = p_ref[0] + p_ref[1] + (1.0 + EPS) * xmi
    g = jnp.dot(m, aw_ref[...], preferred_element_type=jnp.float32)
    h_ref[...] = jax.nn.sigmoid(g) * jax.nn.sigmoid(m)


def _make_epilogue():
    blk = 1000
    grid = (N // blk,)
    return pl.pallas_call(
        _epilogue_body,
        grid=grid,
        in_specs=[pl.BlockSpec((NC, blk, C), lambda i: (0, i, 0)),
                  pl.BlockSpec((blk, C), lambda i: (i, 0)),
                  pl.BlockSpec((C, C), lambda i: (0, 0)),
                  pl.BlockSpec((C, 1), lambda i: (0, 0))],
        out_specs=pl.BlockSpec((blk, C), lambda i: (i, 0)),
        out_shape=jax.ShapeDtypeStruct((N, C), jnp.float32),
    )


@jax.jit
def kernel(x_1, down_laplacian_indices, down_laplacian_values,
           up_laplacian_indices, up_laplacian_values,
           W_down, att_down, W_up, att_up, W_id, att_weight):
    xmd, xmu, sd, td, su, tu = _make_prologue()(
        x_1, W_down, W_up, att_down, att_up)

    tgt_d = down_laplacian_indices[0]
    src_d = down_laplacian_indices[1]
    tgt_u = up_laplacian_indices[0]
    src_u = up_laplacian_indices[1]

    partials = _make_sc_kernel()(
        xmd, xmu,
        src_d, tgt_d, down_laplacian_values,
        src_u, tgt_u, up_laplacian_values,
        sd.reshape(N), td.reshape(N), su.reshape(N), tu.reshape(N))

    return _make_epilogue()(partials, x_1, W_id, att_weight)
